# baseline (speedup 1.0000x reference)
import functools

import jax
import jax.numpy as jnp
from jax import lax
from jax.experimental import pallas as pl
from jax.experimental.pallas import tpu as pltpu
from jax.experimental.pallas import tpu_sc as plsc

N = 8192
D = 768
H = 1536
E = 8
K = 2
EPS = 1e-05

A = N * K
BR = 128
RPAD = A + E * BR
NB = RPAD // BR
BN = 1024

NW = 32
TOK_PER_W = N // NW
DCHUNK = 32
DCH = TOK_PER_W // DCHUNK
TCHUNK = 16
NCH = TOK_PER_W // TCHUNK

_SQRT_HALF = 0.7071067811865476


def _stage1_body(x_ref, lns_ref, lnb_ref, wrt_ref, br_ref,
                 i0_ref, i1_ref, w0_ref, w1_ref):
    xb = x_ref[...]
    mu = jnp.mean(xb, axis=1, keepdims=True)
    var = jnp.mean((xb - mu) ** 2, axis=1, keepdims=True)
    xn = (xb - mu) * lax.rsqrt(var + EPS) * lns_ref[...] + lnb_ref[...]
    logits = jnp.dot(xn, wrt_ref[...], preferred_element_type=jnp.float32)
    logits = logits + br_ref[...]
    col = lax.broadcasted_iota(jnp.int32, logits.shape, 1)
    v0 = jnp.max(logits, axis=1)
    i0 = jnp.argmax(logits, axis=1).astype(jnp.int32)
    neg = jnp.float32(-3.0e38)
    masked = jnp.where(col == i0[:, None], neg, logits)
    v1 = jnp.max(masked, axis=1)
    i1 = jnp.argmax(masked, axis=1).astype(jnp.int32)
    g1 = 1.0 / (1.0 + jnp.exp(v0 - v1))
    i0_ref[...] = i0
    i1_ref[...] = i1
    w0_ref[...] = 1.0 - g1
    w1_ref[...] = g1


def _stage1(x, ln_scale, ln_bias, Wr, br):
    return pl.pallas_call(
        _stage1_body,
        grid=(N // BN,),
        in_specs=[
            pl.BlockSpec((BN, D), lambda b: (b, 0)),
            pl.BlockSpec((1, D), lambda b: (0, 0)),
            pl.BlockSpec((1, D), lambda b: (0, 0)),
            pl.BlockSpec((D, E), lambda b: (0, 0)),
            pl.BlockSpec((1, E), lambda b: (0, 0)),
        ],
        out_specs=[pl.BlockSpec((BN,), lambda b: (b,))] * 4,
        out_shape=[
            jax.ShapeDtypeStruct((N,), jnp.int32),
            jax.ShapeDtypeStruct((N,), jnp.int32),
            jax.ShapeDtypeStruct((N,), jnp.float32),
            jax.ShapeDtypeStruct((N,), jnp.float32),
        ],
    )(x, ln_scale.reshape(1, D), ln_bias.reshape(1, D), Wr.T, br.reshape(1, E))


def _routing_metadata(i0, i1):
    flat_e = jnp.stack([i0, i1], axis=1).reshape(A)
    oh = (flat_e[:, None] == jnp.arange(E, dtype=jnp.int32)[None, :]).astype(jnp.int32)
    csum = jnp.cumsum(oh, axis=0)
    counts = csum[-1]
    cap = ((counts + BR - 1) // BR) * BR
    starts = jnp.concatenate([jnp.zeros((1,), jnp.int32), jnp.cumsum(cap)[:-1].astype(jnp.int32)])
    rank = jnp.sum(csum * oh, axis=1) - 1
    pos_flat = starts[flat_e] + rank
    ends = (starts + cap).astype(jnp.int32)
    bstart = jnp.arange(NB, dtype=jnp.int32) * BR
    blk_expert = jnp.minimum(
        jnp.sum((bstart[:, None] >= ends[None, :]).astype(jnp.int32), axis=1),
        E - 1).astype(jnp.int32)
    pos2 = pos_flat.reshape(N, K)
    return blk_expert, pos2[:, 0], pos2[:, 1]


@functools.lru_cache(maxsize=None)
def _sc_dispatch_fn():
    @functools.partial(
        pl.kernel,
        mesh=plsc.VectorSubcoreMesh(core_axis_name="c", subcore_axis_name="s"),
        out_type=jax.ShapeDtypeStruct((RPAD, D), jnp.float32),
        scratch_types=[
            pltpu.VMEM((DCH, DCHUNK), jnp.int32),
            pltpu.VMEM((DCH, DCHUNK), jnp.int32),
            pltpu.VMEM((2, DCHUNK, D), jnp.float32),
            pltpu.SemaphoreType.DMA,
            pltpu.SemaphoreType.DMA,
            pltpu.SemaphoreType.DMA,
            pltpu.SemaphoreType.DMA,
            pltpu.SemaphoreType.DMA,
            pltpu.SemaphoreType.DMA,
        ],
    )
    def _sc_dispatch(x_hbm, idxe_hbm, idxo_hbm, xg_hbm,
                     idxe_v, idxo_v, rows,
                     sr0, sr1, se0, se1, so0, so1):
        cid = lax.axis_index("c")
        sid = lax.axis_index("s")
        wid = sid * 2 + cid
        sr = (sr0, sr1)
        se = (se0, se1)
        so = (so0, so1)

        pltpu.sync_copy(idxe_hbm.at[wid], idxe_v)
        pltpu.sync_copy(idxo_hbm.at[wid], idxo_v)

        def read(c):
            slot = c % 2
            base = wid * TOK_PER_W + c * DCHUNK
            return pltpu.async_copy(x_hbm.at[pl.ds(base, DCHUNK)],
                                    rows.at[slot], sr[slot])

        rh = {0: read(0)}
        seh = {}
        soh = {}
        for c in range(DCH):
            slot = c % 2
            rh.pop(c).wait()
            seh[c] = pltpu.async_copy(rows.at[slot], xg_hbm.at[idxe_v.at[c]],
                                      se[slot])
            soh[c] = pltpu.async_copy(rows.at[slot], xg_hbm.at[idxo_v.at[c]],
                                      so[slot])
            if c + 1 < DCH:
                if c - 1 >= 0:
                    seh.pop(c - 1).wait()
                    soh.pop(c - 1).wait()
                rh[c + 1] = read(c + 1)
        for h in (*seh.values(), *soh.values()):
            h.wait()

    return _sc_dispatch


def _ffn_body(be_ref, xg_ref, w1_ref, b1_ref, w2_ref, b2_ref,
              lns_ref, lnb_ref, out_ref):
    xb = xg_ref[...]
    mu = jnp.mean(xb, axis=1, keepdims=True)
    var = jnp.mean((xb - mu) ** 2, axis=1, keepdims=True)
    xn = (xb - mu) * lax.rsqrt(var + EPS) * lns_ref[...] + lnb_ref[...]
    h = lax.dot_general(xn, w1_ref[0], (((1,), (1,)), ((), ())),
                        preferred_element_type=jnp.float32)
    h = h + b1_ref[0]
    h = 0.5 * h * (1.0 + lax.erf(h * _SQRT_HALF))
    o = lax.dot_general(h, w2_ref[0], (((1,), (1,)), ((), ())),
                        preferred_element_type=jnp.float32)
    o = o + b2_ref[0]
    out_ref[...] = o


def _ffn(blk_expert, Xg, W1, b1, W2, b2, ln_scale, ln_bias):
    grid_spec = pltpu.PrefetchScalarGridSpec(
        num_scalar_prefetch=1,
        grid=(NB,),
        in_specs=[
            pl.BlockSpec((BR, D), lambda b, be: (b, 0)),
            pl.BlockSpec((1, H, D), lambda b, be: (be[b], 0, 0)),
            pl.BlockSpec((1, 1, H), lambda b, be: (be[b], 0, 0)),
            pl.BlockSpec((1, D, H), lambda b, be: (be[b], 0, 0)),
            pl.BlockSpec((1, 1, D), lambda b, be: (be[b], 0, 0)),
            pl.BlockSpec((1, D), lambda b, be: (0, 0)),
            pl.BlockSpec((1, D), lambda b, be: (0, 0)),
        ],
        out_specs=pl.BlockSpec((BR, D), lambda b, be: (b, 0)),
    )
    return pl.pallas_call(
        _ffn_body,
        grid_spec=grid_spec,
        out_shape=jax.ShapeDtypeStruct((RPAD, D), jnp.float32),
    )(blk_expert, Xg, W1, b1.reshape(E, 1, H),
      W2, b2.reshape(E, 1, D),
      ln_scale.reshape(1, D), ln_bias.reshape(1, D))


@functools.lru_cache(maxsize=None)
def _sc_combine_fn():
    @functools.partial(
        pl.kernel,
        mesh=plsc.VectorSubcoreMesh(core_axis_name="c", subcore_axis_name="s"),
        out_type=jax.ShapeDtypeStruct((N, D), jnp.float32),
        scratch_types=[
            pltpu.VMEM((TOK_PER_W,), jnp.int32),
            pltpu.VMEM((TOK_PER_W,), jnp.int32),
            pltpu.VMEM((TOK_PER_W, 16), jnp.float32),
            pltpu.VMEM((TOK_PER_W, 16), jnp.float32),
            pltpu.VMEM((2, TCHUNK, D), jnp.float32),
            pltpu.VMEM((2, TCHUNK, D), jnp.float32),
            pltpu.SemaphoreType.DMA,
            pltpu.SemaphoreType.DMA,
            pltpu.SemaphoreType.DMA,
            pltpu.SemaphoreType.DMA,
            pltpu.SemaphoreType.DMA,
            pltpu.SemaphoreType.DMA,
        ],
    )
    def _sc_combine(rows_hbm, posa_hbm, posb_hbm, w0_hbm, w1_hbm, y_hbm,
                    ia_v, ib_v, wa_v, wb_v, bufa, bufb,
                    sa0, sa1, sb0, sb1, sw0, sw1):
        wid = lax.axis_index("s") * 2 + lax.axis_index("c")
        base = wid * TOK_PER_W
        sa = (sa0, sa1)
        sb = (sb0, sb1)
        sw = (sw0, sw1)
        pltpu.sync_copy(posa_hbm.at[pl.ds(base, TOK_PER_W)], ia_v)
        pltpu.sync_copy(posb_hbm.at[pl.ds(base, TOK_PER_W)], ib_v)
        pltpu.sync_copy(w0_hbm.at[pl.ds(base, TOK_PER_W)], wa_v)
        pltpu.sync_copy(w1_hbm.at[pl.ds(base, TOK_PER_W)], wb_v)

        def fire(c):
            slot = c % 2
            sl = pl.ds(c * TCHUNK, TCHUNK)
            ha = pltpu.async_copy(rows_hbm.at[ia_v.at[sl]], bufa.at[slot], sa[slot])
            hb = pltpu.async_copy(rows_hbm.at[ib_v.at[sl]], bufb.at[slot], sb[slot])
            return ha, hb

        gh = {0: fire(0)}
        wh = {}
        for c in range(NCH):
            slot = c % 2
            ha, hb = gh.pop(c)
            ha.wait()
            hb.wait()

            @plsc.parallel_loop(0, TCHUNK, unroll=1)
            def add_row(i):
                wa = wa_v[c * TCHUNK + i, :]
                wb = wb_v[c * TCHUNK + i, :]
                for j in range(D // 16):
                    dsl = pl.ds(j * 16, 16)
                    bufa[slot, i, dsl] = (bufa[slot, i, dsl] * wa
                                          + bufb[slot, i, dsl] * wb)

            wh[c] = pltpu.async_copy(
                bufa.at[slot], y_hbm.at[pl.ds(base + c * TCHUNK, TCHUNK)], sw[slot])
            if c + 1 < NCH:
                if c - 1 >= 0:
                    wh.pop(c - 1).wait()
                gh[c + 1] = fire(c + 1)
        for h in wh.values():
            h.wait()

    return _sc_combine


def kernel(x, ln_scale, ln_bias, Wr, br, W1, b1, W2, b2):
    i0, i1, w0, w1 = _stage1(x, ln_scale, ln_bias, Wr, br)
    blk_expert, posa, posb = _routing_metadata(i0, i1)
    idxe = posa.reshape(NW, DCH, DCHUNK)
    idxo = posb.reshape(NW, DCH, DCHUNK)
    xg = _sc_dispatch_fn()(x, idxe, idxo)
    rows = _ffn(blk_expert, xg, W1, b1, W2, b2, ln_scale, ln_bias)
    w0b = jnp.broadcast_to(w0[:, None], (N, 16))
    w1b = jnp.broadcast_to(w1[:, None], (N, 16))
    return _sc_combine_fn()(rows, posa, posb, w0b, w1b)


# speedup vs baseline: 1.5084x; 1.5084x over previous
import functools

import jax
import jax.numpy as jnp
from jax import lax
from jax.experimental import pallas as pl
from jax.experimental.pallas import tpu as pltpu
from jax.experimental.pallas import tpu_sc as plsc

N = 8192
D = 768
H = 1536
E = 8
K = 2
EPS = 1e-05

A = N * K
BR = 512
RPAD = A + E * BR
NB = RPAD // BR
BN = 1024

NW = 32
TOK_PER_W = N // NW
DCHUNK = 32
DCH = TOK_PER_W // DCHUNK
TCHUNK = 16
NCH = TOK_PER_W // TCHUNK

_SQRT_HALF = 0.7071067811865476


def _stage1_body(x_ref, lns_ref, lnb_ref, wrt_ref, br_ref,
                 i0_ref, i1_ref, w0_ref, w1_ref):
    xb = x_ref[...]
    mu = jnp.mean(xb, axis=1, keepdims=True)
    var = jnp.mean((xb - mu) ** 2, axis=1, keepdims=True)
    xn = (xb - mu) * lax.rsqrt(var + EPS) * lns_ref[...] + lnb_ref[...]
    logits = jnp.dot(xn, wrt_ref[...], preferred_element_type=jnp.float32)
    logits = logits + br_ref[...]
    col = lax.broadcasted_iota(jnp.int32, logits.shape, 1)
    v0 = jnp.max(logits, axis=1)
    i0 = jnp.argmax(logits, axis=1).astype(jnp.int32)
    neg = jnp.float32(-3.0e38)
    masked = jnp.where(col == i0[:, None], neg, logits)
    v1 = jnp.max(masked, axis=1)
    i1 = jnp.argmax(masked, axis=1).astype(jnp.int32)
    g1 = 1.0 / (1.0 + jnp.exp(v0 - v1))
    i0_ref[...] = i0
    i1_ref[...] = i1
    w0_ref[...] = 1.0 - g1
    w1_ref[...] = g1


def _stage1(x, ln_scale, ln_bias, Wr, br):
    return pl.pallas_call(
        _stage1_body,
        grid=(N // BN,),
        in_specs=[
            pl.BlockSpec((BN, D), lambda b: (b, 0)),
            pl.BlockSpec((1, D), lambda b: (0, 0)),
            pl.BlockSpec((1, D), lambda b: (0, 0)),
            pl.BlockSpec((D, E), lambda b: (0, 0)),
            pl.BlockSpec((1, E), lambda b: (0, 0)),
        ],
        out_specs=[pl.BlockSpec((BN,), lambda b: (b,))] * 4,
        out_shape=[
            jax.ShapeDtypeStruct((N,), jnp.int32),
            jax.ShapeDtypeStruct((N,), jnp.int32),
            jax.ShapeDtypeStruct((N,), jnp.float32),
            jax.ShapeDtypeStruct((N,), jnp.float32),
        ],
    )(x, ln_scale.reshape(1, D), ln_bias.reshape(1, D), Wr.T, br.reshape(1, E))


def _routing_metadata(i0, i1):
    flat_e = jnp.stack([i0, i1], axis=1).reshape(A)
    oh = (flat_e[:, None] == jnp.arange(E, dtype=jnp.int32)[None, :]).astype(jnp.int32)
    csum = jnp.cumsum(oh, axis=0)
    counts = csum[-1]
    cap = ((counts + BR - 1) // BR) * BR
    starts = jnp.concatenate([jnp.zeros((1,), jnp.int32), jnp.cumsum(cap)[:-1].astype(jnp.int32)])
    rank = jnp.sum(csum * oh, axis=1) - 1
    pos_flat = starts[flat_e] + rank
    ends = (starts + cap).astype(jnp.int32)
    bstart = jnp.arange(NB, dtype=jnp.int32) * BR
    blk_expert = jnp.minimum(
        jnp.sum((bstart[:, None] >= ends[None, :]).astype(jnp.int32), axis=1),
        E - 1).astype(jnp.int32)
    pos2 = pos_flat.reshape(N, K)
    return blk_expert, pos2[:, 0], pos2[:, 1]


@functools.lru_cache(maxsize=None)
def _sc_dispatch_fn():
    @functools.partial(
        pl.kernel,
        mesh=plsc.VectorSubcoreMesh(core_axis_name="c", subcore_axis_name="s"),
        out_type=jax.ShapeDtypeStruct((RPAD, D), jnp.float32),
        scratch_types=[
            pltpu.VMEM((DCH, DCHUNK), jnp.int32),
            pltpu.VMEM((DCH, DCHUNK), jnp.int32),
            pltpu.VMEM((2, DCHUNK, D), jnp.float32),
            pltpu.SemaphoreType.DMA,
            pltpu.SemaphoreType.DMA,
            pltpu.SemaphoreType.DMA,
            pltpu.SemaphoreType.DMA,
            pltpu.SemaphoreType.DMA,
            pltpu.SemaphoreType.DMA,
        ],
    )
    def _sc_dispatch(x_hbm, idxe_hbm, idxo_hbm, xg_hbm,
                     idxe_v, idxo_v, rows,
                     sr0, sr1, se0, se1, so0, so1):
        cid = lax.axis_index("c")
        sid = lax.axis_index("s")
        wid = sid * 2 + cid
        sr = (sr0, sr1)
        se = (se0, se1)
        so = (so0, so1)

        pltpu.sync_copy(idxe_hbm.at[wid], idxe_v)
        pltpu.sync_copy(idxo_hbm.at[wid], idxo_v)

        def read(c):
            slot = c % 2
            base = wid * TOK_PER_W + c * DCHUNK
            return pltpu.async_copy(x_hbm.at[pl.ds(base, DCHUNK)],
                                    rows.at[slot], sr[slot])

        rh = {0: read(0)}
        seh = {}
        soh = {}
        for c in range(DCH):
            slot = c % 2
            rh.pop(c).wait()
            seh[c] = pltpu.async_copy(rows.at[slot], xg_hbm.at[idxe_v.at[c]],
                                      se[slot])
            soh[c] = pltpu.async_copy(rows.at[slot], xg_hbm.at[idxo_v.at[c]],
                                      so[slot])
            if c + 1 < DCH:
                if c - 1 >= 0:
                    seh.pop(c - 1).wait()
                    soh.pop(c - 1).wait()
                rh[c + 1] = read(c + 1)
        for h in (*seh.values(), *soh.values()):
            h.wait()

    return _sc_dispatch


def _ffn_body(be_ref, xg_ref, w1_ref, b1_ref, w2_ref, b2_ref,
              lns_ref, lnb_ref, out_ref):
    xb = xg_ref[...]
    mu = jnp.mean(xb, axis=1, keepdims=True)
    var = jnp.mean((xb - mu) ** 2, axis=1, keepdims=True)
    xn = (xb - mu) * lax.rsqrt(var + EPS) * lns_ref[...] + lnb_ref[...]
    h = lax.dot_general(xn, w1_ref[0], (((1,), (1,)), ((), ())),
                        preferred_element_type=jnp.float32)
    h = h + b1_ref[0]
    h = 0.5 * h * (1.0 + lax.erf(h * _SQRT_HALF))
    o = lax.dot_general(h, w2_ref[0], (((1,), (1,)), ((), ())),
                        preferred_element_type=jnp.float32)
    o = o + b2_ref[0]
    out_ref[...] = o


def _ffn(blk_expert, Xg, W1, b1, W2, b2, ln_scale, ln_bias):
    grid_spec = pltpu.PrefetchScalarGridSpec(
        num_scalar_prefetch=1,
        grid=(NB,),
        in_specs=[
            pl.BlockSpec((BR, D), lambda b, be: (b, 0)),
            pl.BlockSpec((1, H, D), lambda b, be: (be[b], 0, 0)),
            pl.BlockSpec((1, 1, H), lambda b, be: (be[b], 0, 0)),
            pl.BlockSpec((1, D, H), lambda b, be: (be[b], 0, 0)),
            pl.BlockSpec((1, 1, D), lambda b, be: (be[b], 0, 0)),
            pl.BlockSpec((1, D), lambda b, be: (0, 0)),
            pl.BlockSpec((1, D), lambda b, be: (0, 0)),
        ],
        out_specs=pl.BlockSpec((BR, D), lambda b, be: (b, 0)),
    )
    return pl.pallas_call(
        _ffn_body,
        grid_spec=grid_spec,
        out_shape=jax.ShapeDtypeStruct((RPAD, D), jnp.float32),
    )(blk_expert, Xg, W1, b1.reshape(E, 1, H),
      W2, b2.reshape(E, 1, D),
      ln_scale.reshape(1, D), ln_bias.reshape(1, D))


@functools.lru_cache(maxsize=None)
def _sc_combine_fn():
    @functools.partial(
        pl.kernel,
        mesh=plsc.VectorSubcoreMesh(core_axis_name="c", subcore_axis_name="s"),
        out_type=jax.ShapeDtypeStruct((N, D), jnp.float32),
        scratch_types=[
            pltpu.VMEM((TOK_PER_W,), jnp.int32),
            pltpu.VMEM((TOK_PER_W,), jnp.int32),
            pltpu.VMEM((TOK_PER_W, 16), jnp.float32),
            pltpu.VMEM((TOK_PER_W, 16), jnp.float32),
            pltpu.VMEM((2, TCHUNK, D), jnp.float32),
            pltpu.VMEM((2, TCHUNK, D), jnp.float32),
            pltpu.SemaphoreType.DMA,
            pltpu.SemaphoreType.DMA,
            pltpu.SemaphoreType.DMA,
            pltpu.SemaphoreType.DMA,
            pltpu.SemaphoreType.DMA,
            pltpu.SemaphoreType.DMA,
        ],
    )
    def _sc_combine(rows_hbm, posa_hbm, posb_hbm, w0_hbm, w1_hbm, y_hbm,
                    ia_v, ib_v, wa_v, wb_v, bufa, bufb,
                    sa0, sa1, sb0, sb1, sw0, sw1):
        wid = lax.axis_index("s") * 2 + lax.axis_index("c")
        base = wid * TOK_PER_W
        sa = (sa0, sa1)
        sb = (sb0, sb1)
        sw = (sw0, sw1)
        pltpu.sync_copy(posa_hbm.at[pl.ds(base, TOK_PER_W)], ia_v)
        pltpu.sync_copy(posb_hbm.at[pl.ds(base, TOK_PER_W)], ib_v)
        pltpu.sync_copy(w0_hbm.at[pl.ds(base, TOK_PER_W)], wa_v)
        pltpu.sync_copy(w1_hbm.at[pl.ds(base, TOK_PER_W)], wb_v)

        def fire(c):
            slot = c % 2
            sl = pl.ds(c * TCHUNK, TCHUNK)
            ha = pltpu.async_copy(rows_hbm.at[ia_v.at[sl]], bufa.at[slot], sa[slot])
            hb = pltpu.async_copy(rows_hbm.at[ib_v.at[sl]], bufb.at[slot], sb[slot])
            return ha, hb

        gh = {0: fire(0)}
        wh = {}
        for c in range(NCH):
            slot = c % 2
            ha, hb = gh.pop(c)
            ha.wait()
            hb.wait()

            @plsc.parallel_loop(0, TCHUNK, unroll=1)
            def add_row(i):
                wa = wa_v[c * TCHUNK + i, :]
                wb = wb_v[c * TCHUNK + i, :]
                for j in range(D // 16):
                    dsl = pl.ds(j * 16, 16)
                    bufa[slot, i, dsl] = (bufa[slot, i, dsl] * wa
                                          + bufb[slot, i, dsl] * wb)

            wh[c] = pltpu.async_copy(
                bufa.at[slot], y_hbm.at[pl.ds(base + c * TCHUNK, TCHUNK)], sw[slot])
            if c + 1 < NCH:
                if c - 1 >= 0:
                    wh.pop(c - 1).wait()
                gh[c + 1] = fire(c + 1)
        for h in wh.values():
            h.wait()

    return _sc_combine


def kernel(x, ln_scale, ln_bias, Wr, br, W1, b1, W2, b2):
    i0, i1, w0, w1 = _stage1(x, ln_scale, ln_bias, Wr, br)
    blk_expert, posa, posb = _routing_metadata(i0, i1)
    idxe = posa.reshape(NW, DCH, DCHUNK)
    idxo = posb.reshape(NW, DCH, DCHUNK)
    xg = _sc_dispatch_fn()(x, idxe, idxo)
    rows = _ffn(blk_expert, xg, W1, b1, W2, b2, ln_scale, ln_bias)
    w0b = jnp.broadcast_to(w0[:, None], (N, 16))
    w1b = jnp.broadcast_to(w1[:, None], (N, 16))
    return _sc_combine_fn()(rows, posa, posb, w0b, w1b)


# speedup vs baseline: 1.5642x; 1.0370x over previous
import functools

import jax
import jax.numpy as jnp
from jax import lax
from jax.experimental import pallas as pl
from jax.experimental.pallas import tpu as pltpu
from jax.experimental.pallas import tpu_sc as plsc

N = 8192
D = 768
H = 1536
E = 8
K = 2
EPS = 1e-05

A = N * K
BR = 512
RPAD = A + E * BR
NB = RPAD // BR
BN = 1024

NW = 32
TOK_PER_W = N // NW
DCHUNK = 32
DCH = TOK_PER_W // DCHUNK
TCHUNK = 16
NCH = TOK_PER_W // TCHUNK

_SQRT_HALF = 0.7071067811865476


def _stage1_body(x_ref, lns_ref, lnb_ref, wrt_ref, br_ref,
                 i0_ref, i1_ref, w0_ref, w1_ref):
    xb = x_ref[...]
    mu = jnp.mean(xb, axis=1, keepdims=True)
    var = jnp.mean((xb - mu) ** 2, axis=1, keepdims=True)
    xn = (xb - mu) * lax.rsqrt(var + EPS) * lns_ref[...] + lnb_ref[...]
    logits = jnp.dot(xn, wrt_ref[...], preferred_element_type=jnp.float32)
    logits = logits + br_ref[...]
    col = lax.broadcasted_iota(jnp.int32, logits.shape, 1)
    v0 = jnp.max(logits, axis=1)
    i0 = jnp.argmax(logits, axis=1).astype(jnp.int32)
    neg = jnp.float32(-3.0e38)
    masked = jnp.where(col == i0[:, None], neg, logits)
    v1 = jnp.max(masked, axis=1)
    i1 = jnp.argmax(masked, axis=1).astype(jnp.int32)
    g1 = 1.0 / (1.0 + jnp.exp(v0 - v1))
    i0_ref[...] = i0
    i1_ref[...] = i1
    w0_ref[...] = 1.0 - g1
    w1_ref[...] = g1


def _stage1(x, ln_scale, ln_bias, Wr, br):
    return pl.pallas_call(
        _stage1_body,
        grid=(N // BN,),
        in_specs=[
            pl.BlockSpec((BN, D), lambda b: (b, 0)),
            pl.BlockSpec((1, D), lambda b: (0, 0)),
            pl.BlockSpec((1, D), lambda b: (0, 0)),
            pl.BlockSpec((D, E), lambda b: (0, 0)),
            pl.BlockSpec((1, E), lambda b: (0, 0)),
        ],
        out_specs=[pl.BlockSpec((BN,), lambda b: (b,))] * 4,
        out_shape=[
            jax.ShapeDtypeStruct((N,), jnp.int32),
            jax.ShapeDtypeStruct((N,), jnp.int32),
            jax.ShapeDtypeStruct((N,), jnp.float32),
            jax.ShapeDtypeStruct((N,), jnp.float32),
        ],
    )(x, ln_scale.reshape(1, D), ln_bias.reshape(1, D), Wr.T, br.reshape(1, E))


def _routing_metadata(i0, i1):
    eye = jnp.arange(E, dtype=jnp.int32)[None, :]
    oh0 = (i0[:, None] == eye).astype(jnp.int32)
    oh1 = (i1[:, None] == eye).astype(jnp.int32)
    csum = jnp.cumsum(oh0 + oh1, axis=0)
    counts = csum[-1]
    cap = ((counts + BR - 1) // BR) * BR
    starts = jnp.concatenate([jnp.zeros((1,), jnp.int32), jnp.cumsum(cap)[:-1].astype(jnp.int32)])
    posa = starts[i0] + jnp.sum(csum * oh0, axis=1) - 1
    posb = starts[i1] + jnp.sum(csum * oh1, axis=1) - 1
    ends = (starts + cap).astype(jnp.int32)
    bstart = jnp.arange(NB, dtype=jnp.int32) * BR
    blk_expert = jnp.minimum(
        jnp.sum((bstart[:, None] >= ends[None, :]).astype(jnp.int32), axis=1),
        E - 1).astype(jnp.int32)
    return blk_expert, posa, posb


@functools.lru_cache(maxsize=None)
def _sc_dispatch_fn():
    @functools.partial(
        pl.kernel,
        mesh=plsc.VectorSubcoreMesh(core_axis_name="c", subcore_axis_name="s"),
        out_type=jax.ShapeDtypeStruct((RPAD, D), jnp.float32),
        scratch_types=[
            pltpu.VMEM((DCH, DCHUNK), jnp.int32),
            pltpu.VMEM((DCH, DCHUNK), jnp.int32),
            pltpu.VMEM((2, DCHUNK, D), jnp.float32),
            pltpu.SemaphoreType.DMA,
            pltpu.SemaphoreType.DMA,
            pltpu.SemaphoreType.DMA,
            pltpu.SemaphoreType.DMA,
            pltpu.SemaphoreType.DMA,
            pltpu.SemaphoreType.DMA,
        ],
    )
    def _sc_dispatch(x_hbm, idxe_hbm, idxo_hbm, xg_hbm,
                     idxe_v, idxo_v, rows,
                     sr0, sr1, se0, se1, so0, so1):
        cid = lax.axis_index("c")
        sid = lax.axis_index("s")
        wid = sid * 2 + cid
        sr = (sr0, sr1)
        se = (se0, se1)
        so = (so0, so1)

        pltpu.sync_copy(idxe_hbm.at[wid], idxe_v)
        pltpu.sync_copy(idxo_hbm.at[wid], idxo_v)

        def read(c):
            slot = c % 2
            base = wid * TOK_PER_W + c * DCHUNK
            return pltpu.async_copy(x_hbm.at[pl.ds(base, DCHUNK)],
                                    rows.at[slot], sr[slot])

        rh = {0: read(0)}
        seh = {}
        soh = {}
        for c in range(DCH):
            slot = c % 2
            rh.pop(c).wait()
            seh[c] = pltpu.async_copy(rows.at[slot], xg_hbm.at[idxe_v.at[c]],
                                      se[slot])
            soh[c] = pltpu.async_copy(rows.at[slot], xg_hbm.at[idxo_v.at[c]],
                                      so[slot])
            if c + 1 < DCH:
                if c - 1 >= 0:
                    seh.pop(c - 1).wait()
                    soh.pop(c - 1).wait()
                rh[c + 1] = read(c + 1)
        for h in (*seh.values(), *soh.values()):
            h.wait()

    return _sc_dispatch


def _ffn_body(be_ref, xg_ref, w1_ref, b1_ref, w2_ref, b2_ref,
              lns_ref, lnb_ref, out_ref):
    xb = xg_ref[...]
    mu = jnp.mean(xb, axis=1, keepdims=True)
    var = jnp.mean((xb - mu) ** 2, axis=1, keepdims=True)
    xn = (xb - mu) * lax.rsqrt(var + EPS) * lns_ref[...] + lnb_ref[...]
    h = lax.dot_general(xn, w1_ref[0], (((1,), (1,)), ((), ())),
                        preferred_element_type=jnp.float32)
    h = h + b1_ref[0]
    h = 0.5 * h * (1.0 + lax.erf(h * _SQRT_HALF))
    o = lax.dot_general(h, w2_ref[0], (((1,), (1,)), ((), ())),
                        preferred_element_type=jnp.float32)
    o = o + b2_ref[0]
    out_ref[...] = o


def _ffn(blk_expert, Xg, W1, b1, W2, b2, ln_scale, ln_bias):
    grid_spec = pltpu.PrefetchScalarGridSpec(
        num_scalar_prefetch=1,
        grid=(NB,),
        in_specs=[
            pl.BlockSpec((BR, D), lambda b, be: (b, 0)),
            pl.BlockSpec((1, H, D), lambda b, be: (be[b], 0, 0)),
            pl.BlockSpec((1, 1, H), lambda b, be: (be[b], 0, 0)),
            pl.BlockSpec((1, D, H), lambda b, be: (be[b], 0, 0)),
            pl.BlockSpec((1, 1, D), lambda b, be: (be[b], 0, 0)),
            pl.BlockSpec((1, D), lambda b, be: (0, 0)),
            pl.BlockSpec((1, D), lambda b, be: (0, 0)),
        ],
        out_specs=pl.BlockSpec((BR, D), lambda b, be: (b, 0)),
    )
    return pl.pallas_call(
        _ffn_body,
        grid_spec=grid_spec,
        out_shape=jax.ShapeDtypeStruct((RPAD, D), jnp.float32),
    )(blk_expert, Xg, W1, b1.reshape(E, 1, H),
      W2, b2.reshape(E, 1, D),
      ln_scale.reshape(1, D), ln_bias.reshape(1, D))


@functools.lru_cache(maxsize=None)
def _sc_combine_fn():
    @functools.partial(
        pl.kernel,
        mesh=plsc.VectorSubcoreMesh(core_axis_name="c", subcore_axis_name="s"),
        out_type=jax.ShapeDtypeStruct((N, D), jnp.float32),
        scratch_types=[
            pltpu.VMEM((TOK_PER_W,), jnp.int32),
            pltpu.VMEM((TOK_PER_W,), jnp.int32),
            pltpu.VMEM((TOK_PER_W, 16), jnp.float32),
            pltpu.VMEM((TOK_PER_W, 16), jnp.float32),
            pltpu.VMEM((2, TCHUNK, D), jnp.float32),
            pltpu.VMEM((2, TCHUNK, D), jnp.float32),
            pltpu.SemaphoreType.DMA,
            pltpu.SemaphoreType.DMA,
            pltpu.SemaphoreType.DMA,
            pltpu.SemaphoreType.DMA,
            pltpu.SemaphoreType.DMA,
            pltpu.SemaphoreType.DMA,
        ],
    )
    def _sc_combine(rows_hbm, posa_hbm, posb_hbm, w0_hbm, w1_hbm, y_hbm,
                    ia_v, ib_v, wa_v, wb_v, bufa, bufb,
                    sa0, sa1, sb0, sb1, sw0, sw1):
        wid = lax.axis_index("s") * 2 + lax.axis_index("c")
        base = wid * TOK_PER_W
        sa = (sa0, sa1)
        sb = (sb0, sb1)
        sw = (sw0, sw1)
        pltpu.sync_copy(posa_hbm.at[pl.ds(base, TOK_PER_W)], ia_v)
        pltpu.sync_copy(posb_hbm.at[pl.ds(base, TOK_PER_W)], ib_v)
        pltpu.sync_copy(w0_hbm.at[pl.ds(base, TOK_PER_W)], wa_v)
        pltpu.sync_copy(w1_hbm.at[pl.ds(base, TOK_PER_W)], wb_v)

        def fire(c):
            slot = c % 2
            sl = pl.ds(c * TCHUNK, TCHUNK)
            ha = pltpu.async_copy(rows_hbm.at[ia_v.at[sl]], bufa.at[slot], sa[slot])
            hb = pltpu.async_copy(rows_hbm.at[ib_v.at[sl]], bufb.at[slot], sb[slot])
            return ha, hb

        gh = {0: fire(0)}
        wh = {}
        for c in range(NCH):
            slot = c % 2
            ha, hb = gh.pop(c)
            ha.wait()
            hb.wait()

            @plsc.parallel_loop(0, TCHUNK, unroll=1)
            def add_row(i):
                wa = wa_v[c * TCHUNK + i, :]
                wb = wb_v[c * TCHUNK + i, :]
                for j in range(D // 16):
                    dsl = pl.ds(j * 16, 16)
                    bufa[slot, i, dsl] = (bufa[slot, i, dsl] * wa
                                          + bufb[slot, i, dsl] * wb)

            wh[c] = pltpu.async_copy(
                bufa.at[slot], y_hbm.at[pl.ds(base + c * TCHUNK, TCHUNK)], sw[slot])
            if c + 1 < NCH:
                if c - 1 >= 0:
                    wh.pop(c - 1).wait()
                gh[c + 1] = fire(c + 1)
        for h in wh.values():
            h.wait()

    return _sc_combine


def kernel(x, ln_scale, ln_bias, Wr, br, W1, b1, W2, b2):
    i0, i1, w0, w1 = _stage1(x, ln_scale, ln_bias, Wr, br)
    blk_expert, posa, posb = _routing_metadata(i0, i1)
    idxe = posa.reshape(NW, DCH, DCHUNK)
    idxo = posb.reshape(NW, DCH, DCHUNK)
    xg = _sc_dispatch_fn()(x, idxe, idxo)
    rows = _ffn(blk_expert, xg, W1, b1, W2, b2, ln_scale, ln_bias)
    w0b = jnp.broadcast_to(w0[:, None], (N, 16))
    w1b = jnp.broadcast_to(w1[:, None], (N, 16))
    return _sc_combine_fn()(rows, posa, posb, w0b, w1b)
